# baseline (speedup 1.0000x reference)
import functools

import jax
import jax.numpy as jnp
import numpy as np
from jax.experimental import pallas as pl
from jax.experimental.pallas import tpu as pltpu

DIM = 2048
HEADS = 16
HEAD_DIM = 128
N_EXP = 4
TOP_K = 2
H_EXP = 5632
H_SH = 2816
LN_EPS = 1e-6
RMS_EPS = 1e-5


def _layernorm(x):
    m = jnp.mean(x, -1, keepdims=True)
    v = jnp.mean((x - m) ** 2, -1, keepdims=True)
    return (x - m) * jax.lax.rsqrt(v + LN_EPS)


def _qkv_body(x_ref, sc_ref, sh_ref, w_ref, b_ref, qw_ref, kw_ref, o_ref):
    x = x_ref[...]
    xm = _layernorm(x) * (1.0 + sc_ref[...]) + sh_ref[...]
    y = jnp.dot(xm.astype(jnp.bfloat16), w_ref[...],
                preferred_element_type=jnp.float32) + b_ref[...]
    q = y[:, :DIM]
    k = y[:, DIM:2 * DIM]
    v = y[:, 2 * DIM:]
    q = q * jax.lax.rsqrt(jnp.mean(q * q, -1, keepdims=True) + RMS_EPS) * qw_ref[...]
    k = k * jax.lax.rsqrt(jnp.mean(k * k, -1, keepdims=True) + RMS_EPS) * kw_ref[...]
    o_ref[...] = jnp.concatenate([q, k, v], axis=1)


def _qkv(x, sc, sh, w, b, qw, kw, bm):
    s = x.shape[0]
    return pl.pallas_call(
        _qkv_body,
        grid=(s // bm,),
        in_specs=[
            pl.BlockSpec((bm, DIM), lambda i: (i, 0)),
            pl.BlockSpec((1, DIM), lambda i: (0, 0)),
            pl.BlockSpec((1, DIM), lambda i: (0, 0)),
            pl.BlockSpec((DIM, 3 * DIM), lambda i: (0, 0)),
            pl.BlockSpec((1, 3 * DIM), lambda i: (0, 0)),
            pl.BlockSpec((1, DIM), lambda i: (0, 0)),
            pl.BlockSpec((1, DIM), lambda i: (0, 0)),
        ],
        out_specs=pl.BlockSpec((bm, 3 * DIM), lambda i: (i, 0)),
        out_shape=jax.ShapeDtypeStruct((s, 3 * DIM), jnp.float32),
    )(x, sc, sh, w, b, qw, kw)


def _rope_ab(x, a, b):
    even = jax.lax.broadcasted_iota(jnp.int32, x.shape, 1) % 2 == 0
    swap = jnp.where(even, jnp.roll(x, -1, axis=-1), jnp.roll(x, 1, axis=-1))
    return a * x + b * swap


def _attn_body(qi_ref, qt_ref, ki_ref, kt_ref, vi_ref, vt_ref,
               aq_ref, bq_ref, ak_ref, bk_ref, o_ref, k_sc, v_sc, *, nqi):
    i = pl.program_id(1)

    @pl.when(i == 0)
    def _():
        k = jnp.concatenate([ki_ref[...], kt_ref[...]], axis=0)
        k_sc[...] = _rope_ab(k, ak_ref[...], bk_ref[...]).astype(jnp.bfloat16)
        v_sc[...] = jnp.concatenate(
            [vi_ref[...], vt_ref[...]], axis=0).astype(jnp.bfloat16)

    q_raw = jnp.where(i < nqi, qi_ref[...], qt_ref[...])
    q = _rope_ab(q_raw, aq_ref[...], bq_ref[...]).astype(jnp.bfloat16)
    s = jax.lax.dot_general(q, k_sc[...], (((1,), (1,)), ((), ())),
                            preferred_element_type=jnp.float32)
    m = jnp.max(s, -1, keepdims=True)
    p = jnp.exp(s - m)
    l = jnp.sum(p, -1, keepdims=True)
    o = jnp.dot(p.astype(jnp.bfloat16), v_sc[...],
                preferred_element_type=jnp.float32)
    o_ref[...] = o / l


def _attention(qkv_i, qkv_t, aq, bq, ak, bk, bm):
    si = qkv_i.shape[0]
    st = qkv_t.shape[0]
    seq = si + st
    nqi = si // bm
    nq = nqi + 1
    d = HEAD_DIM
    return pl.pallas_call(
        functools.partial(_attn_body, nqi=nqi),
        grid=(HEADS, nq),
        in_specs=[
            pl.BlockSpec((bm, d), lambda hh, i: (jnp.minimum(i, nqi - 1), hh)),
            pl.BlockSpec((st, d), lambda hh, i: (0, hh)),
            pl.BlockSpec((si, d), lambda hh, i: (0, HEADS + hh)),
            pl.BlockSpec((st, d), lambda hh, i: (0, HEADS + hh)),
            pl.BlockSpec((si, d), lambda hh, i: (0, 2 * HEADS + hh)),
            pl.BlockSpec((st, d), lambda hh, i: (0, 2 * HEADS + hh)),
            pl.BlockSpec((bm, d), lambda hh, i: (i, 0)),
            pl.BlockSpec((bm, d), lambda hh, i: (i, 0)),
            pl.BlockSpec((seq, d), lambda hh, i: (0, 0)),
            pl.BlockSpec((seq, d), lambda hh, i: (0, 0)),
        ],
        out_specs=pl.BlockSpec((bm, d), lambda hh, i: (i, hh)),
        out_shape=jax.ShapeDtypeStruct((seq, DIM), jnp.float32),
        scratch_shapes=[pltpu.VMEM((seq, d), jnp.bfloat16),
                        pltpu.VMEM((seq, d), jnp.bfloat16)],
        compiler_params=pltpu.CompilerParams(
            dimension_semantics=("arbitrary", "arbitrary")),
    )(qkv_i, qkv_t, qkv_i, qkv_t, qkv_i, qkv_t, aq, bq, ak, bk)


def _outproj_body(a_ref, w_ref, b_ref, g_ref, r_ref, o_ref):
    y = jnp.dot(a_ref[...].astype(jnp.bfloat16),
                w_ref[...].astype(jnp.bfloat16),
                preferred_element_type=jnp.float32) + b_ref[...]
    o_ref[...] = r_ref[...] + g_ref[...] * y


def _outproj(a, w, b, g, res, bm):
    s = a.shape[0]
    return pl.pallas_call(
        _outproj_body,
        grid=(s // bm,),
        in_specs=[
            pl.BlockSpec((bm, DIM), lambda i: (i, 0)),
            pl.BlockSpec((DIM, DIM), lambda i: (0, 0)),
            pl.BlockSpec((1, DIM), lambda i: (0, 0)),
            pl.BlockSpec((1, DIM), lambda i: (0, 0)),
            pl.BlockSpec((bm, DIM), lambda i: (i, 0)),
        ],
        out_specs=pl.BlockSpec((bm, DIM), lambda i: (i, 0)),
        out_shape=jax.ShapeDtypeStruct((s, DIM), jnp.float32),
    )(a, w, b, g, res)


def _lnmod_body(x_ref, sc_ref, sh_ref, gw_ref, z_ref, zb_ref, gl_ref):
    x = x_ref[...]
    z = _layernorm(x) * (1.0 + sc_ref[...]) + sh_ref[...]
    z_ref[...] = z
    zb = z.astype(jnp.bfloat16)
    zb_ref[...] = zb
    gl_ref[...] = jax.lax.dot_general(
        z, gw_ref[...], (((1,), (1,)), ((), ())),
        preferred_element_type=jnp.float32)


def _lnmod(x, sc, sh, gate_w_pad, bm):
    s = x.shape[0]
    ne = gate_w_pad.shape[0]
    return pl.pallas_call(
        _lnmod_body,
        grid=(s // bm,),
        in_specs=[
            pl.BlockSpec((bm, DIM), lambda i: (i, 0)),
            pl.BlockSpec((1, DIM), lambda i: (0, 0)),
            pl.BlockSpec((1, DIM), lambda i: (0, 0)),
            pl.BlockSpec((ne, DIM), lambda i: (0, 0)),
        ],
        out_specs=[
            pl.BlockSpec((bm, DIM), lambda i: (i, 0)),
            pl.BlockSpec((bm, DIM), lambda i: (i, 0)),
            pl.BlockSpec((bm, ne), lambda i: (i, 0)),
        ],
        out_shape=[
            jax.ShapeDtypeStruct((s, DIM), jnp.float32),
            jax.ShapeDtypeStruct((s, DIM), jnp.bfloat16),
            jax.ShapeDtypeStruct((s, ne), jnp.float32),
        ],
    )(x, sc, sh, gate_w_pad)


def _swiglu_body(z_ref, w1_ref, w3_ref, w2_ref, g_ref, rs_ref, r_ref, o_ref,
                 acc_ref, *, nh):
    j = pl.program_id(1)
    z = z_ref[...]
    a = jnp.dot(z, w1_ref[...], preferred_element_type=jnp.float32)
    c = jnp.dot(z, w3_ref[...], preferred_element_type=jnp.float32)
    h = (a * jax.nn.sigmoid(a)) * c
    part = jnp.dot(h.astype(jnp.bfloat16), w2_ref[...],
                   preferred_element_type=jnp.float32)

    @pl.when(j == 0)
    def _():
        acc_ref[...] = part

    @pl.when(j > 0)
    def _():
        acc_ref[...] += part

    @pl.when(j == nh - 1)
    def _():
        o_ref[...] = r_ref[...] + g_ref[...] * (rs_ref[...] * acc_ref[...])


def _swiglu(zb, w1, w3, w2, g, rowscale, res, bm, bh):
    s = zb.shape[0]
    hdim = w1.shape[1]
    nh = hdim // bh
    return pl.pallas_call(
        functools.partial(_swiglu_body, nh=nh),
        grid=(s // bm, nh),
        in_specs=[
            pl.BlockSpec((bm, DIM), lambda i, j: (i, 0)),
            pl.BlockSpec((DIM, bh), lambda i, j: (0, j)),
            pl.BlockSpec((DIM, bh), lambda i, j: (0, j)),
            pl.BlockSpec((bh, DIM), lambda i, j: (j, 0)),
            pl.BlockSpec((1, DIM), lambda i, j: (0, 0)),
            pl.BlockSpec((bm, 1), lambda i, j: (i, 0)),
            pl.BlockSpec((bm, DIM), lambda i, j: (i, 0)),
        ],
        out_specs=pl.BlockSpec((bm, DIM), lambda i, j: (i, 0)),
        out_shape=jax.ShapeDtypeStruct((s, DIM), jnp.float32),
        scratch_shapes=[pltpu.VMEM((bm, DIM), jnp.float32)],
        compiler_params=pltpu.CompilerParams(
            dimension_semantics=("parallel", "arbitrary")),
    )(zb, w1, w3, w2, g, rowscale, res)


def _moe_body(z_ref, w1_ref, w3_ref, w2_ref, g_ref, rs_ref, r_ref, o_ref,
              acc_ref, *, ne, nh):
    e = pl.program_id(1)
    j = pl.program_id(2)
    z = z_ref[...]
    a = jnp.dot(z, w1_ref[0], preferred_element_type=jnp.float32)
    c = jnp.dot(z, w3_ref[0], preferred_element_type=jnp.float32)
    h = (a * jax.nn.sigmoid(a)) * c
    part = jnp.dot(h.astype(jnp.bfloat16), w2_ref[0],
                   preferred_element_type=jnp.float32)
    part = rs_ref[0] * part

    @pl.when((e == 0) & (j == 0))
    def _():
        acc_ref[...] = part

    @pl.when((e > 0) | (j > 0))
    def _():
        acc_ref[...] += part

    @pl.when((e == ne - 1) & (j == nh - 1))
    def _():
        o_ref[...] = r_ref[...] + g_ref[...] * acc_ref[...]


def _moe(zb, w1, w3, w2, g, rowscale, res, bm, bh):
    s = zb.shape[0]
    ne, _, hdim = w1.shape
    nh = hdim // bh
    return pl.pallas_call(
        functools.partial(_moe_body, ne=ne, nh=nh),
        grid=(s // bm, ne, nh),
        in_specs=[
            pl.BlockSpec((bm, DIM), lambda i, e, j: (i, 0)),
            pl.BlockSpec((1, DIM, bh), lambda i, e, j: (e, 0, j)),
            pl.BlockSpec((1, DIM, bh), lambda i, e, j: (e, 0, j)),
            pl.BlockSpec((1, bh, DIM), lambda i, e, j: (e, j, 0)),
            pl.BlockSpec((1, DIM), lambda i, e, j: (0, 0)),
            pl.BlockSpec((1, bm, 1), lambda i, e, j: (e, i, 0)),
            pl.BlockSpec((bm, DIM), lambda i, e, j: (i, 0)),
        ],
        out_specs=pl.BlockSpec((bm, DIM), lambda i, e, j: (i, 0)),
        out_shape=jax.ShapeDtypeStruct((s, DIM), jnp.float32),
        scratch_shapes=[pltpu.VMEM((bm, DIM), jnp.float32)],
        compiler_params=pltpu.CompilerParams(
            dimension_semantics=("parallel", "arbitrary", "arbitrary")),
    )(zb, w1, w3, w2, g, rowscale, res)


MOE_BM = 512
MOE_MAXT = (2 * 2048) // MOE_BM + N_EXP - 1


def _moe_hidden_body(te_ref, z_ref, w1_ref, w3_ref, o_ref):
    i = pl.program_id(1)
    z = z_ref[pl.ds(i * MOE_BM, MOE_BM), :]
    w1 = w1_ref[0].astype(jnp.bfloat16)
    w3 = w3_ref[0].astype(jnp.bfloat16)
    a = jnp.dot(z, w1, preferred_element_type=jnp.float32)
    c = jnp.dot(z, w3, preferred_element_type=jnp.float32)
    o_ref[...] = ((a * jax.nn.sigmoid(a)) * c).astype(jnp.bfloat16)


def _moe_hidden(tile_expert, z_sorted, w1, w3, bh):
    rows = z_sorted.shape[0]
    nh = H_EXP // bh
    grid_spec = pltpu.PrefetchScalarGridSpec(
        num_scalar_prefetch=1,
        grid=(nh, MOE_MAXT),
        in_specs=[
            pl.BlockSpec((rows, DIM), lambda j, i, te: (0, 0)),
            pl.BlockSpec((1, DIM, bh), lambda j, i, te: (te[i], 0, j)),
            pl.BlockSpec((1, DIM, bh), lambda j, i, te: (te[i], 0, j)),
        ],
        out_specs=pl.BlockSpec((MOE_BM, bh), lambda j, i, te: (i, j)),
    )
    return pl.pallas_call(
        _moe_hidden_body,
        grid_spec=grid_spec,
        out_shape=jax.ShapeDtypeStruct((rows, H_EXP), jnp.bfloat16),
        compiler_params=pltpu.CompilerParams(
            dimension_semantics=("arbitrary", "arbitrary")),
    )(tile_expert, z_sorted, w1, w3)


def _moe_down_body(te_ref, h_ref, w2_ref, o_ref):
    o_ref[...] = jnp.dot(h_ref[...], w2_ref[0].astype(jnp.bfloat16),
                         preferred_element_type=jnp.float32)


def _moe_down(tile_expert, h_sorted, w2, bc):
    rows = h_sorted.shape[0]
    nc = DIM // bc
    grid_spec = pltpu.PrefetchScalarGridSpec(
        num_scalar_prefetch=1,
        grid=(nc, MOE_MAXT),
        in_specs=[
            pl.BlockSpec((MOE_BM, H_EXP), lambda c, i, te: (i, 0)),
            pl.BlockSpec((1, H_EXP, bc), lambda c, i, te: (te[i], 0, c)),
        ],
        out_specs=pl.BlockSpec((MOE_BM, bc), lambda c, i, te: (i, c)),
    )
    return pl.pallas_call(
        _moe_down_body,
        grid_spec=grid_spec,
        out_shape=jax.ShapeDtypeStruct((rows, DIM), jnp.float32),
        compiler_params=pltpu.CompilerParams(
            dimension_semantics=("arbitrary", "arbitrary")),
    )(tile_expert, h_sorted, w2)


def _mm_body(x_ref, w_ref, b_ref, o_ref):
    o_ref[...] = jnp.dot(x_ref[...].astype(jnp.bfloat16),
                         w_ref[...].astype(jnp.bfloat16),
                         preferred_element_type=jnp.float32) + b_ref[...]


def _mm(x, w, b, bn):
    m, k = x.shape
    n = w.shape[1]
    return pl.pallas_call(
        _mm_body,
        grid=(n // bn,),
        in_specs=[
            pl.BlockSpec((m, k), lambda j: (0, 0)),
            pl.BlockSpec((k, bn), lambda j: (0, j)),
            pl.BlockSpec((1, bn), lambda j: (0, j)),
        ],
        out_specs=pl.BlockSpec((m, bn), lambda j: (0, j)),
        out_shape=jax.ShapeDtypeStruct((m, n), jnp.float32),
    )(x, w, b)


def _apply_rope(xq, xk, freqs):
    xq_ = xq.reshape(*xq.shape[:-1], -1, 1, 2)
    xk_ = xk.reshape(*xk.shape[:-1], -1, 1, 2)
    xq_o = freqs[..., 0] * xq_[..., 0] + freqs[..., 1] * xq_[..., 1]
    xk_o = freqs[..., 0] * xk_[..., 0] + freqs[..., 1] * xk_[..., 1]
    return xq_o.reshape(xq.shape), xk_o.reshape(xk.shape)


def kernel(image_tokens, text_tokens, adaln_input, rope, params):
    p = params
    b, img_len, _ = image_tokens.shape
    txt_len = text_tokens.shape[1]
    seq = img_len + txt_len
    bf = jnp.bfloat16

    silu_in = jax.nn.silu(adaln_input)
    mod = _mm(silu_in, p["adaln"]["w"], p["adaln"]["b"][None, :], bn=2048)
    (sh_mi, sc_mi, g_mi, sh_fi, sc_fi, g_fi,
     sh_mt, sc_mt, g_mt, sh_ft, sc_ft, g_ft) = jnp.split(mod, 12, axis=-1)

    xi = image_tokens[0]
    xt = text_tokens[0]

    wqkv_i = jnp.concatenate(
        [p["to_q"]["w"], p["to_k"]["w"], p["to_v"]["w"]], axis=1).astype(bf)
    bqkv_i = jnp.concatenate(
        [p["to_q"]["b"], p["to_k"]["b"], p["to_v"]["b"]])[None, :]
    wqkv_t = jnp.concatenate(
        [p["to_q_t"]["w"], p["to_k_t"]["w"], p["to_v_t"]["w"]], axis=1).astype(bf)
    bqkv_t = jnp.concatenate(
        [p["to_q_t"]["b"], p["to_k_t"]["b"], p["to_v_t"]["b"]])[None, :]

    qkv_i = _qkv(xi, sc_mi, sh_mi, wqkv_i, bqkv_i,
                 p["q_rms"][None, :], p["k_rms"][None, :], bm=256)
    qkv_t = _qkv(xt, sc_mt, sh_mt, wqkv_t, bqkv_t,
                 p["q_rms_t"][None, :], p["k_rms_t"][None, :], bm=256)

    f = rope[0, :, 0]
    ra = jnp.stack([f[:, :, 0, 0], f[:, :, 1, 1]], -1).reshape(seq, HEAD_DIM)
    rb = jnp.stack([f[:, :, 0, 1], f[:, :, 1, 0]], -1).reshape(seq, HEAD_DIM)
    scale = 1.0 / np.sqrt(HEAD_DIM)
    attn = _attention(qkv_i, qkv_t, ra * scale, rb * scale, ra, rb, bm=256)

    ai, at = attn[:img_len], attn[img_len:]
    xi = _outproj(ai, p["to_out"]["w"], p["to_out"]["b"][None, :],
                  g_mi, xi, bm=256)
    xt = _outproj(at, p["to_out_t"]["w"], p["to_out_t"]["b"][None, :],
                  g_mt, xt, bm=256)

    gate_w = p["gate_w"]
    z_i, zb_i, logits = _lnmod(xi, sc_fi, sh_fi, gate_w, bm=256)
    del z_i
    _, zb_t, _ = _lnmod(xt, sc_ft, sh_ft, gate_w, bm=256)

    scores = jax.nn.softmax(logits, axis=-1)
    rank = jnp.sum(
        (scores[:, None, :] > scores[:, :, None])
        | ((scores[:, None, :] == scores[:, :, None])
           & (jnp.arange(N_EXP)[None, :] < jnp.arange(N_EXP)[:, None])[None]),
        axis=-1)
    keep = rank < TOP_K
    wfull = jnp.where(keep, scores, 0.0)

    ones_rs = jnp.ones((txt_len, 1), jnp.float32)
    xt = _swiglu(zb_t, p["t_w1"].astype(bf), p["t_w3"].astype(bf),
                 p["t_w2"].astype(bf), g_ft, ones_rs, xt, bm=256, bh=1408)

    acc = _swiglu(zb_i, p["sh_w1"].astype(bf), p["sh_w3"].astype(bf),
                  p["sh_w2"].astype(bf), g_fi, jnp.ones((img_len, 1), jnp.float32),
                  xi, bm=256, bh=2816)
    t = img_len
    bmm = MOE_BM
    flat_keep = keep.T.reshape(-1)
    flat_tok = jnp.tile(jnp.arange(t, dtype=jnp.int32), N_EXP)
    flat_e = jnp.repeat(jnp.arange(N_EXP, dtype=jnp.int32), t)
    flat_w = wfull.T.reshape(-1)
    sort_key = flat_e + N_EXP * (1 - flat_keep.astype(jnp.int32))
    order = jnp.argsort(sort_key, stable=True)
    c = jnp.sum(keep, axis=0).astype(jnp.int32)
    start = jnp.concatenate([jnp.zeros((1,), jnp.int32),
                             jnp.cumsum(c)])[:N_EXP]
    ntile_e = (c + bmm - 1) // bmm
    pad_off = jnp.concatenate([jnp.zeros((1,), jnp.int32),
                               jnp.cumsum(ntile_e)]).astype(jnp.int32)
    rows = MOE_MAXT * bmm
    row_ids = jnp.arange(rows, dtype=jnp.int32)
    tile_of_row = row_ids // bmm
    e_of_row = jnp.clip(
        jnp.searchsorted(pad_off, tile_of_row, side="right") - 1, 0, N_EXP - 1
    ).astype(jnp.int32)
    r_in_e = row_ids - pad_off[e_of_row] * bmm
    valid = r_in_e < c[e_of_row]
    src = start[e_of_row] + jnp.minimum(r_in_e, jnp.maximum(c[e_of_row] - 1, 0))
    src_flat = order[src]
    tok_padded = jnp.where(valid, flat_tok[src_flat], 0)
    w_padded = jnp.where(valid, flat_w[src_flat], 0.0)
    tile_expert = jnp.clip(
        jnp.searchsorted(pad_off, jnp.arange(MOE_MAXT, dtype=jnp.int32),
                         side="right") - 1, 0, N_EXP - 1).astype(jnp.int32)

    z_sorted = jnp.take(zb_i, tok_padded, axis=0)
    h_sorted = _moe_hidden(tile_expert, z_sorted,
                           p["exp_w1"], p["exp_w3"], bh=512)
    eo_sorted = _moe_down(tile_expert, h_sorted, p["exp_w2"], bc=512)

    rank = (jnp.cumsum(keep, axis=0) - 1).astype(jnp.int32)
    prow = pad_off[None, :N_EXP] * bmm + rank
    p1 = jnp.min(jnp.where(keep, prow, rows), axis=1)
    p2 = jnp.max(jnp.where(keep, prow, -1), axis=1)
    y = (w_padded[p1, None] * jnp.take(eo_sorted, p1, axis=0)
         + w_padded[p2, None] * jnp.take(eo_sorted, p2, axis=0))
    xi = acc + g_fi * y

    return xi[None], xt[None]


# speedup vs baseline: 1.0146x; 1.0146x over previous
import functools

import jax
import jax.numpy as jnp
import numpy as np
from jax.experimental import pallas as pl
from jax.experimental.pallas import tpu as pltpu

DIM = 2048
HEADS = 16
HEAD_DIM = 128
N_EXP = 4
TOP_K = 2
H_EXP = 5632
H_SH = 2816
LN_EPS = 1e-6
RMS_EPS = 1e-5


def _layernorm(x):
    m = jnp.mean(x, -1, keepdims=True)
    v = jnp.mean((x - m) ** 2, -1, keepdims=True)
    return (x - m) * jax.lax.rsqrt(v + LN_EPS)


def _qkv_body(x_ref, sc_ref, sh_ref, w_ref, b_ref, qw_ref, kw_ref, o_ref):
    x = x_ref[...]
    xm = _layernorm(x) * (1.0 + sc_ref[...]) + sh_ref[...]
    y = jnp.dot(xm.astype(jnp.bfloat16), w_ref[...],
                preferred_element_type=jnp.float32) + b_ref[...]
    q = y[:, :DIM]
    k = y[:, DIM:2 * DIM]
    v = y[:, 2 * DIM:]
    q = q * jax.lax.rsqrt(jnp.mean(q * q, -1, keepdims=True) + RMS_EPS) * qw_ref[...]
    k = k * jax.lax.rsqrt(jnp.mean(k * k, -1, keepdims=True) + RMS_EPS) * kw_ref[...]
    o_ref[...] = jnp.concatenate([q, k, v], axis=1)


def _qkv(x, sc, sh, w, b, qw, kw, bm):
    s = x.shape[0]
    return pl.pallas_call(
        _qkv_body,
        grid=(s // bm,),
        in_specs=[
            pl.BlockSpec((bm, DIM), lambda i: (i, 0)),
            pl.BlockSpec((1, DIM), lambda i: (0, 0)),
            pl.BlockSpec((1, DIM), lambda i: (0, 0)),
            pl.BlockSpec((DIM, 3 * DIM), lambda i: (0, 0)),
            pl.BlockSpec((1, 3 * DIM), lambda i: (0, 0)),
            pl.BlockSpec((1, DIM), lambda i: (0, 0)),
            pl.BlockSpec((1, DIM), lambda i: (0, 0)),
        ],
        out_specs=pl.BlockSpec((bm, 3 * DIM), lambda i: (i, 0)),
        out_shape=jax.ShapeDtypeStruct((s, 3 * DIM), jnp.float32),
    )(x, sc, sh, w, b, qw, kw)


def _rope_ab(x, a, b):
    even = jax.lax.broadcasted_iota(jnp.int32, x.shape, 1) % 2 == 0
    swap = jnp.where(even, jnp.roll(x, -1, axis=-1), jnp.roll(x, 1, axis=-1))
    return a * x + b * swap


def _attn_body(qi_ref, qt_ref, ki_ref, kt_ref, vi_ref, vt_ref,
               aq_ref, bq_ref, ak_ref, bk_ref, o_ref, k_sc, v_sc, *, nqi):
    i = pl.program_id(1)

    @pl.when(i == 0)
    def _():
        k = jnp.concatenate([ki_ref[...], kt_ref[...]], axis=0)
        k_sc[...] = _rope_ab(k, ak_ref[...], bk_ref[...]).astype(jnp.bfloat16)
        v_sc[...] = jnp.concatenate(
            [vi_ref[...], vt_ref[...]], axis=0).astype(jnp.bfloat16)

    q_raw = jnp.where(i < nqi, qi_ref[...], qt_ref[...])
    q = _rope_ab(q_raw, aq_ref[...], bq_ref[...]).astype(jnp.bfloat16)
    s = jax.lax.dot_general(q, k_sc[...], (((1,), (1,)), ((), ())),
                            preferred_element_type=jnp.float32)
    m = jnp.max(s, -1, keepdims=True)
    p = jnp.exp(s - m)
    l = jnp.sum(p, -1, keepdims=True)
    o = jnp.dot(p.astype(jnp.bfloat16), v_sc[...],
                preferred_element_type=jnp.float32)
    o_ref[...] = o / l


def _attention(qkv_i, qkv_t, aq, bq, ak, bk, bm):
    si = qkv_i.shape[0]
    st = qkv_t.shape[0]
    seq = si + st
    nqi = si // bm
    nq = nqi + 1
    d = HEAD_DIM
    return pl.pallas_call(
        functools.partial(_attn_body, nqi=nqi),
        grid=(HEADS, nq),
        in_specs=[
            pl.BlockSpec((bm, d), lambda hh, i: (jnp.minimum(i, nqi - 1), hh)),
            pl.BlockSpec((st, d), lambda hh, i: (0, hh)),
            pl.BlockSpec((si, d), lambda hh, i: (0, HEADS + hh)),
            pl.BlockSpec((st, d), lambda hh, i: (0, HEADS + hh)),
            pl.BlockSpec((si, d), lambda hh, i: (0, 2 * HEADS + hh)),
            pl.BlockSpec((st, d), lambda hh, i: (0, 2 * HEADS + hh)),
            pl.BlockSpec((bm, d), lambda hh, i: (i, 0)),
            pl.BlockSpec((bm, d), lambda hh, i: (i, 0)),
            pl.BlockSpec((seq, d), lambda hh, i: (0, 0)),
            pl.BlockSpec((seq, d), lambda hh, i: (0, 0)),
        ],
        out_specs=pl.BlockSpec((bm, d), lambda hh, i: (i, hh)),
        out_shape=jax.ShapeDtypeStruct((seq, DIM), jnp.float32),
        scratch_shapes=[pltpu.VMEM((seq, d), jnp.bfloat16),
                        pltpu.VMEM((seq, d), jnp.bfloat16)],
        compiler_params=pltpu.CompilerParams(
            dimension_semantics=("arbitrary", "arbitrary")),
    )(qkv_i, qkv_t, qkv_i, qkv_t, qkv_i, qkv_t, aq, bq, ak, bk)


def _outproj_body(a_ref, w_ref, b_ref, g_ref, r_ref, o_ref):
    y = jnp.dot(a_ref[...].astype(jnp.bfloat16),
                w_ref[...].astype(jnp.bfloat16),
                preferred_element_type=jnp.float32) + b_ref[...]
    o_ref[...] = r_ref[...] + g_ref[...] * y


def _outproj(a, w, b, g, res, bm, row0=0):
    s = res.shape[0]
    return pl.pallas_call(
        _outproj_body,
        grid=(s // bm,),
        in_specs=[
            pl.BlockSpec((bm, DIM), lambda i: (row0 + i, 0)),
            pl.BlockSpec((DIM, DIM), lambda i: (0, 0)),
            pl.BlockSpec((1, DIM), lambda i: (0, 0)),
            pl.BlockSpec((1, DIM), lambda i: (0, 0)),
            pl.BlockSpec((bm, DIM), lambda i: (i, 0)),
        ],
        out_specs=pl.BlockSpec((bm, DIM), lambda i: (i, 0)),
        out_shape=jax.ShapeDtypeStruct((s, DIM), jnp.float32),
    )(a, w, b, g, res)


def _lnmod_body(x_ref, sc_ref, sh_ref, gw_ref, zb_ref, gl_ref):
    x = x_ref[...]
    z = _layernorm(x) * (1.0 + sc_ref[...]) + sh_ref[...]
    zb_ref[...] = z.astype(jnp.bfloat16)
    gl_ref[...] = jax.lax.dot_general(
        z, gw_ref[...], (((1,), (1,)), ((), ())),
        preferred_element_type=jnp.float32)


def _lnmod(x, sc, sh, gate_w_pad, bm):
    s = x.shape[0]
    ne = gate_w_pad.shape[0]
    return pl.pallas_call(
        _lnmod_body,
        grid=(s // bm,),
        in_specs=[
            pl.BlockSpec((bm, DIM), lambda i: (i, 0)),
            pl.BlockSpec((1, DIM), lambda i: (0, 0)),
            pl.BlockSpec((1, DIM), lambda i: (0, 0)),
            pl.BlockSpec((ne, DIM), lambda i: (0, 0)),
        ],
        out_specs=[
            pl.BlockSpec((bm, DIM), lambda i: (i, 0)),
            pl.BlockSpec((bm, ne), lambda i: (i, 0)),
        ],
        out_shape=[
            jax.ShapeDtypeStruct((s, DIM), jnp.bfloat16),
            jax.ShapeDtypeStruct((s, ne), jnp.float32),
        ],
    )(x, sc, sh, gate_w_pad)


def _swiglu_body(z_ref, w1_ref, w3_ref, w2_ref, g_ref, rs_ref, r_ref, o_ref,
                 acc_ref, *, nh):
    j = pl.program_id(1)
    z = z_ref[...]
    a = jnp.dot(z, w1_ref[...], preferred_element_type=jnp.float32)
    c = jnp.dot(z, w3_ref[...], preferred_element_type=jnp.float32)
    h = (a * jax.nn.sigmoid(a)) * c
    part = jnp.dot(h.astype(jnp.bfloat16), w2_ref[...],
                   preferred_element_type=jnp.float32)

    @pl.when(j == 0)
    def _():
        acc_ref[...] = part

    @pl.when(j > 0)
    def _():
        acc_ref[...] += part

    @pl.when(j == nh - 1)
    def _():
        o_ref[...] = r_ref[...] + g_ref[...] * (rs_ref[...] * acc_ref[...])


def _swiglu(zb, w1, w3, w2, g, rowscale, res, bm, bh):
    s = zb.shape[0]
    hdim = w1.shape[1]
    nh = hdim // bh
    return pl.pallas_call(
        functools.partial(_swiglu_body, nh=nh),
        grid=(s // bm, nh),
        in_specs=[
            pl.BlockSpec((bm, DIM), lambda i, j: (i, 0)),
            pl.BlockSpec((DIM, bh), lambda i, j: (0, j)),
            pl.BlockSpec((DIM, bh), lambda i, j: (0, j)),
            pl.BlockSpec((bh, DIM), lambda i, j: (j, 0)),
            pl.BlockSpec((1, DIM), lambda i, j: (0, 0)),
            pl.BlockSpec((bm, 1), lambda i, j: (i, 0)),
            pl.BlockSpec((bm, DIM), lambda i, j: (i, 0)),
        ],
        out_specs=pl.BlockSpec((bm, DIM), lambda i, j: (i, 0)),
        out_shape=jax.ShapeDtypeStruct((s, DIM), jnp.float32),
        scratch_shapes=[pltpu.VMEM((bm, DIM), jnp.float32)],
        compiler_params=pltpu.CompilerParams(
            dimension_semantics=("parallel", "arbitrary")),
    )(zb, w1, w3, w2, g, rowscale, res)


def _moe_body(z_ref, w1_ref, w3_ref, w2_ref, g_ref, rs_ref, r_ref, o_ref,
              acc_ref, *, ne, nh):
    e = pl.program_id(1)
    j = pl.program_id(2)
    z = z_ref[...]
    a = jnp.dot(z, w1_ref[0], preferred_element_type=jnp.float32)
    c = jnp.dot(z, w3_ref[0], preferred_element_type=jnp.float32)
    h = (a * jax.nn.sigmoid(a)) * c
    part = jnp.dot(h.astype(jnp.bfloat16), w2_ref[0],
                   preferred_element_type=jnp.float32)
    part = rs_ref[0] * part

    @pl.when((e == 0) & (j == 0))
    def _():
        acc_ref[...] = part

    @pl.when((e > 0) | (j > 0))
    def _():
        acc_ref[...] += part

    @pl.when((e == ne - 1) & (j == nh - 1))
    def _():
        o_ref[...] = r_ref[...] + g_ref[...] * acc_ref[...]


def _moe(zb, w1, w3, w2, g, rowscale, res, bm, bh):
    s = zb.shape[0]
    ne, _, hdim = w1.shape
    nh = hdim // bh
    return pl.pallas_call(
        functools.partial(_moe_body, ne=ne, nh=nh),
        grid=(s // bm, ne, nh),
        in_specs=[
            pl.BlockSpec((bm, DIM), lambda i, e, j: (i, 0)),
            pl.BlockSpec((1, DIM, bh), lambda i, e, j: (e, 0, j)),
            pl.BlockSpec((1, DIM, bh), lambda i, e, j: (e, 0, j)),
            pl.BlockSpec((1, bh, DIM), lambda i, e, j: (e, j, 0)),
            pl.BlockSpec((1, DIM), lambda i, e, j: (0, 0)),
            pl.BlockSpec((1, bm, 1), lambda i, e, j: (e, i, 0)),
            pl.BlockSpec((bm, DIM), lambda i, e, j: (i, 0)),
        ],
        out_specs=pl.BlockSpec((bm, DIM), lambda i, e, j: (i, 0)),
        out_shape=jax.ShapeDtypeStruct((s, DIM), jnp.float32),
        scratch_shapes=[pltpu.VMEM((bm, DIM), jnp.float32)],
        compiler_params=pltpu.CompilerParams(
            dimension_semantics=("parallel", "arbitrary", "arbitrary")),
    )(zb, w1, w3, w2, g, rowscale, res)


MOE_BM = 256
MOE_MAXT = (2 * 2048) // MOE_BM + N_EXP - 1


def _moe_hidden_body(te_ref, z_ref, w1_ref, w3_ref, o_ref):
    i = pl.program_id(1)
    z = z_ref[pl.ds(i * MOE_BM, MOE_BM), :]
    w1 = w1_ref[0].astype(jnp.bfloat16)
    w3 = w3_ref[0].astype(jnp.bfloat16)
    a = jnp.dot(z, w1, preferred_element_type=jnp.float32)
    c = jnp.dot(z, w3, preferred_element_type=jnp.float32)
    o_ref[...] = ((a * jax.nn.sigmoid(a)) * c).astype(jnp.bfloat16)


def _moe_hidden(tile_expert, z_sorted, w1, w3, bh):
    rows = z_sorted.shape[0]
    nh = H_EXP // bh
    grid_spec = pltpu.PrefetchScalarGridSpec(
        num_scalar_prefetch=1,
        grid=(nh, MOE_MAXT),
        in_specs=[
            pl.BlockSpec((rows, DIM), lambda j, i, te: (0, 0)),
            pl.BlockSpec((1, DIM, bh), lambda j, i, te: (te[i], 0, j)),
            pl.BlockSpec((1, DIM, bh), lambda j, i, te: (te[i], 0, j)),
        ],
        out_specs=pl.BlockSpec((MOE_BM, bh), lambda j, i, te: (i, j)),
    )
    return pl.pallas_call(
        _moe_hidden_body,
        grid_spec=grid_spec,
        out_shape=jax.ShapeDtypeStruct((rows, H_EXP), jnp.bfloat16),
        compiler_params=pltpu.CompilerParams(
            dimension_semantics=("arbitrary", "arbitrary")),
    )(tile_expert, z_sorted, w1, w3)


def _moe_down_body(te_ref, h_ref, w2_ref, o_ref):
    o_ref[...] = jnp.dot(h_ref[...], w2_ref[0].astype(jnp.bfloat16),
                         preferred_element_type=jnp.float32)


def _moe_down(tile_expert, h_sorted, w2, bc):
    rows = h_sorted.shape[0]
    nc = DIM // bc
    grid_spec = pltpu.PrefetchScalarGridSpec(
        num_scalar_prefetch=1,
        grid=(nc, MOE_MAXT),
        in_specs=[
            pl.BlockSpec((MOE_BM, H_EXP), lambda c, i, te: (i, 0)),
            pl.BlockSpec((1, H_EXP, bc), lambda c, i, te: (te[i], 0, c)),
        ],
        out_specs=pl.BlockSpec((MOE_BM, bc), lambda c, i, te: (i, c)),
    )
    return pl.pallas_call(
        _moe_down_body,
        grid_spec=grid_spec,
        out_shape=jax.ShapeDtypeStruct((rows, DIM), jnp.float32),
        compiler_params=pltpu.CompilerParams(
            dimension_semantics=("arbitrary", "arbitrary")),
    )(tile_expert, h_sorted, w2)


def _mm_body(x_ref, w_ref, b_ref, o_ref):
    o_ref[...] = jnp.dot(x_ref[...].astype(jnp.bfloat16),
                         w_ref[...].astype(jnp.bfloat16),
                         preferred_element_type=jnp.float32) + b_ref[...]


def _mm(x, w, b, bn):
    m, k = x.shape
    n = w.shape[1]
    return pl.pallas_call(
        _mm_body,
        grid=(n // bn,),
        in_specs=[
            pl.BlockSpec((m, k), lambda j: (0, 0)),
            pl.BlockSpec((k, bn), lambda j: (0, j)),
            pl.BlockSpec((1, bn), lambda j: (0, j)),
        ],
        out_specs=pl.BlockSpec((m, bn), lambda j: (0, j)),
        out_shape=jax.ShapeDtypeStruct((m, n), jnp.float32),
    )(x, w, b)


def _apply_rope(xq, xk, freqs):
    xq_ = xq.reshape(*xq.shape[:-1], -1, 1, 2)
    xk_ = xk.reshape(*xk.shape[:-1], -1, 1, 2)
    xq_o = freqs[..., 0] * xq_[..., 0] + freqs[..., 1] * xq_[..., 1]
    xk_o = freqs[..., 0] * xk_[..., 0] + freqs[..., 1] * xk_[..., 1]
    return xq_o.reshape(xq.shape), xk_o.reshape(xk.shape)


def kernel(image_tokens, text_tokens, adaln_input, rope, params):
    p = params
    b, img_len, _ = image_tokens.shape
    txt_len = text_tokens.shape[1]
    seq = img_len + txt_len
    bf = jnp.bfloat16

    silu_in = jax.nn.silu(adaln_input)
    mod = _mm(silu_in, p["adaln"]["w"], p["adaln"]["b"][None, :], bn=2048)
    (sh_mi, sc_mi, g_mi, sh_fi, sc_fi, g_fi,
     sh_mt, sc_mt, g_mt, sh_ft, sc_ft, g_ft) = jnp.split(mod, 12, axis=-1)

    xi = image_tokens[0]
    xt = text_tokens[0]

    wqkv_i = jnp.concatenate(
        [p["to_q"]["w"].astype(bf), p["to_k"]["w"].astype(bf),
         p["to_v"]["w"].astype(bf)], axis=1)
    bqkv_i = jnp.concatenate(
        [p["to_q"]["b"], p["to_k"]["b"], p["to_v"]["b"]])[None, :]
    wqkv_t = jnp.concatenate(
        [p["to_q_t"]["w"].astype(bf), p["to_k_t"]["w"].astype(bf),
         p["to_v_t"]["w"].astype(bf)], axis=1)
    bqkv_t = jnp.concatenate(
        [p["to_q_t"]["b"], p["to_k_t"]["b"], p["to_v_t"]["b"]])[None, :]

    qkv_i = _qkv(xi, sc_mi, sh_mi, wqkv_i, bqkv_i,
                 p["q_rms"][None, :], p["k_rms"][None, :], bm=256)
    qkv_t = _qkv(xt, sc_mt, sh_mt, wqkv_t, bqkv_t,
                 p["q_rms_t"][None, :], p["k_rms_t"][None, :], bm=256)

    f = rope[0, :, 0]
    ra = jnp.stack([f[:, :, 0, 0], f[:, :, 1, 1]], -1).reshape(seq, HEAD_DIM)
    rb = jnp.stack([f[:, :, 0, 1], f[:, :, 1, 0]], -1).reshape(seq, HEAD_DIM)
    scale = 1.0 / np.sqrt(HEAD_DIM)
    attn = _attention(qkv_i, qkv_t, ra * scale, rb * scale, ra, rb, bm=256)

    xi = _outproj(attn, p["to_out"]["w"], p["to_out"]["b"][None, :],
                  g_mi, xi, bm=256, row0=0)
    xt = _outproj(attn, p["to_out_t"]["w"], p["to_out_t"]["b"][None, :],
                  g_mt, xt, bm=256, row0=img_len // 256)

    gate_w = p["gate_w"]
    zb_i, logits = _lnmod(xi, sc_fi, sh_fi, gate_w, bm=256)
    zb_t, _ = _lnmod(xt, sc_ft, sh_ft, gate_w, bm=256)

    scores = jax.nn.softmax(logits, axis=-1)
    rank = jnp.sum(
        (scores[:, None, :] > scores[:, :, None])
        | ((scores[:, None, :] == scores[:, :, None])
           & (jnp.arange(N_EXP)[None, :] < jnp.arange(N_EXP)[:, None])[None]),
        axis=-1)
    keep = rank < TOP_K
    wfull = jnp.where(keep, scores, 0.0)

    ones_rs = jnp.ones((txt_len, 1), jnp.float32)
    xt = _swiglu(zb_t, p["t_w1"].astype(bf), p["t_w3"].astype(bf),
                 p["t_w2"].astype(bf), g_ft, ones_rs, xt, bm=256, bh=1408)

    acc = _swiglu(zb_i, p["sh_w1"].astype(bf), p["sh_w3"].astype(bf),
                  p["sh_w2"].astype(bf), g_fi, jnp.ones((img_len, 1), jnp.float32),
                  xi, bm=256, bh=2816)
    t = img_len
    bmm = MOE_BM
    flat_keep = keep.T.reshape(-1)
    flat_tok = jnp.tile(jnp.arange(t, dtype=jnp.int32), N_EXP)
    flat_e = jnp.repeat(jnp.arange(N_EXP, dtype=jnp.int32), t)
    flat_w = wfull.T.reshape(-1)
    sort_key = flat_e + N_EXP * (1 - flat_keep.astype(jnp.int32))
    order = jnp.argsort(sort_key, stable=True)
    c = jnp.sum(keep, axis=0).astype(jnp.int32)
    start = jnp.concatenate([jnp.zeros((1,), jnp.int32),
                             jnp.cumsum(c)])[:N_EXP]
    ntile_e = (c + bmm - 1) // bmm
    pad_off = jnp.concatenate([jnp.zeros((1,), jnp.int32),
                               jnp.cumsum(ntile_e)]).astype(jnp.int32)
    rows = MOE_MAXT * bmm
    row_ids = jnp.arange(rows, dtype=jnp.int32)
    tile_of_row = row_ids // bmm
    e_of_row = jnp.clip(
        jnp.searchsorted(pad_off, tile_of_row, side="right") - 1, 0, N_EXP - 1
    ).astype(jnp.int32)
    r_in_e = row_ids - pad_off[e_of_row] * bmm
    valid = r_in_e < c[e_of_row]
    src = start[e_of_row] + jnp.minimum(r_in_e, jnp.maximum(c[e_of_row] - 1, 0))
    src_flat = order[src]
    tok_padded = jnp.where(valid, flat_tok[src_flat], 0)
    w_padded = jnp.where(valid, flat_w[src_flat], 0.0)
    tile_expert = jnp.clip(
        jnp.searchsorted(pad_off, jnp.arange(MOE_MAXT, dtype=jnp.int32),
                         side="right") - 1, 0, N_EXP - 1).astype(jnp.int32)

    z_sorted = jnp.take(zb_i, tok_padded, axis=0)
    h_sorted = _moe_hidden(tile_expert, z_sorted,
                           p["exp_w1"], p["exp_w3"], bh=512)
    eo_sorted = _moe_down(tile_expert, h_sorted, p["exp_w2"], bc=1024)

    rank = (jnp.cumsum(keep, axis=0) - 1).astype(jnp.int32)
    prow = pad_off[None, :N_EXP] * bmm + rank
    p1 = jnp.min(jnp.where(keep, prow, rows), axis=1)
    p2 = jnp.max(jnp.where(keep, prow, -1), axis=1)
    y = (w_padded[p1, None] * jnp.take(eo_sorted, p1, axis=0)
         + w_padded[p2, None] * jnp.take(eo_sorted, p2, axis=0))
    xi = acc + g_fi * y

    return xi[None], xt[None]
